# baseline (speedup 1.0000x reference)
import jax
import jax.numpy as jnp
from jax import lax
from jax.experimental import pallas as pl
from jax.experimental.pallas import tpu as pltpu
from jax.experimental.pallas import tpu_sc as plsc

_NUM_ANCS = 9
_PRE = 6000
_POST = 300
_IOU_T = 0.7
_HW = 4096
_N = _HW * _NUM_ANCS
_ROWS = _N // 128
_TM = 512
_NW = 32
_CHUNK = _N // _NW
_CAP = 6144
_CROWS = _CAP // 128


def _mm_body(x_ref, w1_ref, b1_ref, wh_ref, bh_ref, y_ref):
    h = jnp.dot(x_ref[...], w1_ref[...], preferred_element_type=jnp.float32)
    h = jnp.maximum(h + b1_ref[...], 0.0)
    y = jnp.dot(h, wh_ref[...], preferred_element_type=jnp.float32) + bh_ref[...]
    lane = lax.broadcasted_iota(jnp.int32, (_TM, 128), 1)
    y_ref[...] = jnp.where(lane < 16, jax.nn.sigmoid(y), y)


def _prep_body(s_ref, a0_ref, a1_ref, a2_ref, a3_ref, r0_ref, r1_ref, r2_ref,
               r3_ref, x1_ref, y1_ref, x2_ref, y2_ref, dst_ref):
    a2 = a2_ref[...]
    a3 = a3_ref[...]
    cx = a0_ref[...] + r0_ref[...] * a2
    cy = a1_ref[...] + r1_ref[...] * a3
    w = a2 * jnp.exp(r2_ref[...])
    h = a3 * jnp.exp(r3_ref[...])
    x1_ref[...] = jnp.clip(cx - w * 0.5, 0.0, 1.0)
    y1_ref[...] = jnp.clip(cy - h * 0.5, 0.0, 1.0)
    x2_ref[...] = jnp.clip(cx + w * 0.5, 0.0, 1.0)
    y2_ref[...] = jnp.clip(cy + h * 0.5, 0.0, 1.0)

    bits = lax.bitcast_convert_type(s_ref[...], jnp.int32)
    idx = (lax.broadcasted_iota(jnp.int32, (_ROWS, 128), 0) * 128
           + lax.broadcasted_iota(jnp.int32, (_ROWS, 128), 1))
    tau = jnp.int32(0)
    for b in range(30, -1, -1):
        cand = tau | jnp.int32(1 << b)
        cnt = jnp.sum((bits >= cand).astype(jnp.int32))
        tau = jnp.where(cnt >= _PRE, cand, tau)
    cnt_gt = jnp.sum((bits > tau).astype(jnp.int32))
    tie = bits == tau
    need = _PRE - cnt_gt
    theta = jnp.int32(0)
    for b in range(15, -1, -1):
        cand = theta | jnp.int32(1 << b)
        cnt = jnp.sum((tie & (idx >= cand)).astype(jnp.int32))
        theta = jnp.where(cnt >= need, cand, theta)
    active = (bits > tau) | (tie & (idx >= theta))

    act_f = jnp.where(active, 1.0, 0.0)
    tri_l = (lax.broadcasted_iota(jnp.int32, (_ROWS, _ROWS), 0)
             > lax.broadcasted_iota(jnp.int32, (_ROWS, _ROWS), 1)).astype(jnp.float32)
    tri_u = (lax.broadcasted_iota(jnp.int32, (128, 128), 0)
             < lax.broadcasted_iota(jnp.int32, (128, 128), 1)).astype(jnp.float32)
    row_sum = jnp.sum(act_f, axis=1, keepdims=True)
    row_pre = jnp.dot(tri_l, row_sum, preferred_element_type=jnp.float32)
    lane_pre = jnp.dot(act_f, tri_u, preferred_element_type=jnp.float32)
    prefix = (row_pre + lane_pre).astype(jnp.int32)
    dst_ref[...] = jnp.where(active, prefix, _CAP + idx)


def _sc_compact_body(s_hbm, x1_hbm, y1_hbm, x2_hbm, y2_hbm, dst_hbm,
                     os_hbm, ox1_hbm, oy1_hbm, ox2_hbm, oy2_hbm,
                     sb, xb1, yb1, xb2, yb2, idx_v, sem):
    wid = lax.axis_index("s") * 2 + lax.axis_index("c")
    base_in = wid * _CHUNK
    rows = _CHUNK // 128
    pltpu.sync_copy(s_hbm.at[pl.ds(base_in, _CHUNK)], sb)
    pltpu.sync_copy(x1_hbm.at[pl.ds(base_in, _CHUNK)], xb1)
    pltpu.sync_copy(y1_hbm.at[pl.ds(base_in, _CHUNK)], yb1)
    pltpu.sync_copy(x2_hbm.at[pl.ds(base_in, _CHUNK)], xb2)
    pltpu.sync_copy(y2_hbm.at[pl.ds(base_in, _CHUNK)], yb2)
    for r in range(rows):
        pltpu.sync_copy(dst_hbm.at[pl.ds(base_in + r * 128, 128)], idx_v.at[r])

    copies = []
    for buf, out in ((sb, os_hbm), (xb1, ox1_hbm), (yb1, oy1_hbm),
                     (xb2, ox2_hbm), (yb2, oy2_hbm)):
        for r in range(rows):
            copies.append(pltpu.async_copy(
                buf.at[pl.ds(r * 128, 128)], out.at[idx_v.at[r]], sem))
    for c in copies:
        c.wait()


def _nms_body(s_ref, x1_ref, y1_ref, x2_ref, y2_ref,
              ox1_ref, oy1_ref, ox2_ref, oy2_ref, ms_ref, ar_ref):
    pos = (lax.broadcasted_iota(jnp.int32, (_CROWS, 128), 0) * 128
           + lax.broadcasted_iota(jnp.int32, (_CROWS, 128), 1))
    ms_ref[...] = jnp.where(pos < _PRE, s_ref[...], -1.0)
    x1 = x1_ref[...]
    y1 = y1_ref[...]
    x2 = x2_ref[...]
    y2 = y2_ref[...]
    ar_ref[...] = (x2 - x1) * (y2 - y1)

    def zero_body(k, c):
        ox1_ref[k] = 0.0
        oy1_ref[k] = 0.0
        ox2_ref[k] = 0.0
        oy2_ref[k] = 0.0
        return c

    lax.fori_loop(0, _POST, zero_body, 0)

    def nms_step(k, c):
        ms = ms_ref[...]
        m = jnp.max(ms)

        @pl.when(m >= 0.0)
        def _():
            bi = jnp.max(jnp.where(ms == m, pos, -1))
            br_ = bi // 128
            bc_ = bi % 128
            lane_eq = lax.broadcasted_iota(jnp.int32, (1, 128), 1) == bc_

            def _pick(ref):
                return jnp.sum(jnp.where(lane_eq, ref[pl.ds(br_, 1), :], 0.0))

            bx1 = _pick(x1_ref)
            by1 = _pick(y1_ref)
            bx2 = _pick(x2_ref)
            by2 = _pick(y2_ref)
            ix1 = jnp.maximum(x1_ref[...], bx1)
            iy1 = jnp.maximum(y1_ref[...], by1)
            ix2 = jnp.minimum(x2_ref[...], bx2)
            iy2 = jnp.minimum(y2_ref[...], by2)
            inter = (jnp.maximum(ix2 - ix1, 0.0) * jnp.maximum(iy2 - iy1, 0.0))
            barea = (bx2 - bx1) * (by2 - by1)
            union = jnp.maximum(barea + ar_ref[...] - inter, 1e-8)
            supp = inter > _IOU_T * union
            ms_ref[...] = jnp.where(supp, -1.0, ms)
            ox1_ref[k] = bx1
            oy1_ref[k] = by1
            ox2_ref[k] = bx2
            oy2_ref[k] = by2

        return c

    lax.fori_loop(0, _POST, nms_step, 0)


def kernel(feats, ancs, ancs_valid, W1, b1, Wc, bc, Wr, br):
    del ancs_valid
    x = feats.reshape(_HW, 1024)

    wh = jnp.zeros((512, 128), jnp.float32)
    bh = jnp.zeros((128,), jnp.float32)
    wh = wh.at[:, 0:_NUM_ANCS].set(Wc)
    bh = bh.at[0:_NUM_ANCS].set(bc)
    for c in range(4):
        wh = wh.at[:, 16 * (c + 1):16 * (c + 1) + _NUM_ANCS].set(Wr[:, c::4])
        bh = bh.at[16 * (c + 1):16 * (c + 1) + _NUM_ANCS].set(br[c::4])

    y = pl.pallas_call(
        _mm_body,
        grid=(_HW // _TM,),
        in_specs=[
            pl.BlockSpec((_TM, 1024), lambda i: (i, 0)),
            pl.BlockSpec((1024, 512), lambda i: (0, 0)),
            pl.BlockSpec((1, 512), lambda i: (0, 0)),
            pl.BlockSpec((512, 128), lambda i: (0, 0)),
            pl.BlockSpec((1, 128), lambda i: (0, 0)),
        ],
        out_specs=pl.BlockSpec((_TM, 128), lambda i: (i, 0)),
        out_shape=jax.ShapeDtypeStruct((_HW, 128), jnp.float32),
    )(x, W1, b1.reshape(1, 512), wh, bh.reshape(1, 128))

    cls_pred = y[:, 0:_NUM_ANCS]
    reg_planes = [y[:, 16 * (c + 1):16 * (c + 1) + _NUM_ANCS] for c in range(4)]

    scores = cls_pred.reshape(_ROWS, 128)
    ancs_flat = ancs.reshape(_N, 4)
    anc_planes = [ancs_flat[:, c].reshape(_ROWS, 128) for c in range(4)]
    reg2 = [p.reshape(_ROWS, 128) for p in reg_planes]

    plane = jax.ShapeDtypeStruct((_ROWS, 128), jnp.float32)
    x1p, y1p, x2p, y2p, dst = pl.pallas_call(
        _prep_body,
        in_specs=[pl.BlockSpec(memory_space=pltpu.VMEM)] * 9,
        out_specs=[pl.BlockSpec(memory_space=pltpu.VMEM)] * 5,
        out_shape=[plane] * 4 + [jax.ShapeDtypeStruct((_ROWS, 128), jnp.int32)],
    )(scores, *anc_planes, *reg2)

    mesh = plsc.VectorSubcoreMesh(core_axis_name="c", subcore_axis_name="s")
    flat = jax.ShapeDtypeStruct((_CAP + _N,), jnp.float32)
    sc_compact = pl.kernel(
        _sc_compact_body, mesh=mesh,
        out_type=[flat] * 5,
        scratch_types=(
            [pltpu.VMEM((_CHUNK,), jnp.float32)] * 5
            + [pltpu.VMEM((_CHUNK // 128, 128), jnp.int32),
               pltpu.SemaphoreType.DMA]
        ),
    )
    comp = sc_compact(scores.reshape(_N), x1p.reshape(_N), y1p.reshape(_N),
                      x2p.reshape(_N), y2p.reshape(_N), dst.reshape(_N))
    cplanes = [a[:_CAP].reshape(_CROWS, 128) for a in comp]

    outs = pl.pallas_call(
        _nms_body,
        in_specs=[pl.BlockSpec(memory_space=pltpu.VMEM)] * 5,
        out_specs=[pl.BlockSpec(memory_space=pltpu.SMEM)] * 4,
        out_shape=[jax.ShapeDtypeStruct((_POST,), jnp.float32)] * 4,
        scratch_shapes=[pltpu.VMEM((_CROWS, 128), jnp.float32)] * 2,
    )(*cplanes)

    proposals = jnp.stack(outs, axis=-1)
    cls_out = cls_pred.reshape(1, 64, 64, _NUM_ANCS)
    reg_out = jnp.stack(reg_planes, axis=-1).reshape(1, 64, 64, _NUM_ANCS, 4)
    return cls_out, reg_out, proposals


# speedup vs baseline: 1.0069x; 1.0069x over previous
import jax
import jax.numpy as jnp
from jax import lax
from jax.experimental import pallas as pl
from jax.experimental.pallas import tpu as pltpu
from jax.experimental.pallas import tpu_sc as plsc

_NUM_ANCS = 9
_PRE = 6000
_POST = 300
_IOU_T = 0.7
_HW = 4096
_N = _HW * _NUM_ANCS
_ROWS = _N // 128
_TM = 512
_NW = 32
_CHUNK = _N // _NW
_CAP = 6144
_CROWS = _CAP // 128


def _mm_body(x_ref, w1_ref, b1_ref, wh_ref, bh_ref, y_ref):
    h = jnp.dot(x_ref[...], w1_ref[...], preferred_element_type=jnp.float32)
    h = jnp.maximum(h + b1_ref[...], 0.0)
    y = jnp.dot(h, wh_ref[...], preferred_element_type=jnp.float32) + bh_ref[...]
    lane = lax.broadcasted_iota(jnp.int32, (_TM, 128), 1)
    y_ref[...] = jnp.where(lane < 16, jax.nn.sigmoid(y), y)


def _prep_body(s_ref, a0_ref, a1_ref, a2_ref, a3_ref, r0_ref, r1_ref, r2_ref,
               r3_ref, x1_ref, y1_ref, x2_ref, y2_ref, dst_ref):
    a2 = a2_ref[...]
    a3 = a3_ref[...]
    cx = a0_ref[...] + r0_ref[...] * a2
    cy = a1_ref[...] + r1_ref[...] * a3
    w = a2 * jnp.exp(r2_ref[...])
    h = a3 * jnp.exp(r3_ref[...])
    x1_ref[...] = jnp.clip(cx - w * 0.5, 0.0, 1.0)
    y1_ref[...] = jnp.clip(cy - h * 0.5, 0.0, 1.0)
    x2_ref[...] = jnp.clip(cx + w * 0.5, 0.0, 1.0)
    y2_ref[...] = jnp.clip(cy + h * 0.5, 0.0, 1.0)

    bits = lax.bitcast_convert_type(s_ref[...], jnp.int32)
    idx = (lax.broadcasted_iota(jnp.int32, (_ROWS, 128), 0) * 128
           + lax.broadcasted_iota(jnp.int32, (_ROWS, 128), 1))
    tau = jnp.int32(0)
    for b in range(30, -1, -1):
        cand = tau | jnp.int32(1 << b)
        cnt = jnp.sum((bits >= cand).astype(jnp.int32))
        tau = jnp.where(cnt >= _PRE, cand, tau)
    cnt_gt = jnp.sum((bits > tau).astype(jnp.int32))
    tie = bits == tau
    need = _PRE - cnt_gt
    theta = jnp.int32(0)
    for b in range(15, -1, -1):
        cand = theta | jnp.int32(1 << b)
        cnt = jnp.sum((tie & (idx >= cand)).astype(jnp.int32))
        theta = jnp.where(cnt >= need, cand, theta)
    active = (bits > tau) | (tie & (idx >= theta))

    act_f = jnp.where(active, 1.0, 0.0)
    tri_l = (lax.broadcasted_iota(jnp.int32, (_ROWS, _ROWS), 0)
             > lax.broadcasted_iota(jnp.int32, (_ROWS, _ROWS), 1)).astype(jnp.float32)
    tri_u = (lax.broadcasted_iota(jnp.int32, (128, 128), 0)
             < lax.broadcasted_iota(jnp.int32, (128, 128), 1)).astype(jnp.float32)
    row_sum = jnp.sum(act_f, axis=1, keepdims=True)
    row_pre = jnp.dot(tri_l, row_sum, preferred_element_type=jnp.float32)
    lane_pre = jnp.dot(act_f, tri_u, preferred_element_type=jnp.float32)
    prefix = (row_pre + lane_pre).astype(jnp.int32)
    dst_ref[...] = jnp.where(active, prefix, _CAP + idx)


def _sc_compact_body(s_hbm, x1_hbm, y1_hbm, x2_hbm, y2_hbm, dst_hbm,
                     os_hbm, ox1_hbm, oy1_hbm, ox2_hbm, oy2_hbm,
                     sb, xb1, yb1, xb2, yb2, idx_v, sem):
    wid = lax.axis_index("s") * 2 + lax.axis_index("c")
    base_in = wid * _CHUNK
    rows = _CHUNK // 128
    pltpu.sync_copy(s_hbm.at[pl.ds(base_in, _CHUNK)], sb)
    pltpu.sync_copy(x1_hbm.at[pl.ds(base_in, _CHUNK)], xb1)
    pltpu.sync_copy(y1_hbm.at[pl.ds(base_in, _CHUNK)], yb1)
    pltpu.sync_copy(x2_hbm.at[pl.ds(base_in, _CHUNK)], xb2)
    pltpu.sync_copy(y2_hbm.at[pl.ds(base_in, _CHUNK)], yb2)
    pltpu.sync_copy(dst_hbm.at[pl.ds(base_in, _CHUNK)], idx_v)

    copies = []
    for buf, out in ((sb, os_hbm), (xb1, ox1_hbm), (yb1, oy1_hbm),
                     (xb2, ox2_hbm), (yb2, oy2_hbm)):
        copies.append(pltpu.async_copy(buf, out.at[idx_v], sem))
    for c in copies:
        c.wait()


def _nms_body(s_ref, x1_ref, y1_ref, x2_ref, y2_ref,
              ox1_ref, oy1_ref, ox2_ref, oy2_ref, ms_ref, ar_ref):
    pos = (lax.broadcasted_iota(jnp.int32, (_CROWS, 128), 0) * 128
           + lax.broadcasted_iota(jnp.int32, (_CROWS, 128), 1))
    ms_ref[...] = jnp.where(pos < _PRE, s_ref[...], -1.0)
    x1 = x1_ref[...]
    y1 = y1_ref[...]
    x2 = x2_ref[...]
    y2 = y2_ref[...]
    ar_ref[...] = (x2 - x1) * (y2 - y1)

    def zero_body(k, c):
        ox1_ref[k] = 0.0
        oy1_ref[k] = 0.0
        ox2_ref[k] = 0.0
        oy2_ref[k] = 0.0
        return c

    lax.fori_loop(0, _POST, zero_body, 0)

    def nms_step(k, c):
        ms = ms_ref[...]
        m = jnp.max(ms)

        @pl.when(m >= 0.0)
        def _():
            bi = jnp.max(jnp.where(ms == m, pos, -1))
            br_ = bi // 128
            bc_ = bi % 128
            lane_eq = lax.broadcasted_iota(jnp.int32, (1, 128), 1) == bc_

            def _pick(ref):
                return jnp.sum(jnp.where(lane_eq, ref[pl.ds(br_, 1), :], 0.0))

            bx1 = _pick(x1_ref)
            by1 = _pick(y1_ref)
            bx2 = _pick(x2_ref)
            by2 = _pick(y2_ref)
            ix1 = jnp.maximum(x1_ref[...], bx1)
            iy1 = jnp.maximum(y1_ref[...], by1)
            ix2 = jnp.minimum(x2_ref[...], bx2)
            iy2 = jnp.minimum(y2_ref[...], by2)
            inter = (jnp.maximum(ix2 - ix1, 0.0) * jnp.maximum(iy2 - iy1, 0.0))
            barea = (bx2 - bx1) * (by2 - by1)
            union = jnp.maximum(barea + ar_ref[...] - inter, 1e-8)
            supp = inter > _IOU_T * union
            ms_ref[...] = jnp.where(supp, -1.0, ms)
            ox1_ref[k] = bx1
            oy1_ref[k] = by1
            ox2_ref[k] = bx2
            oy2_ref[k] = by2

        return c

    lax.fori_loop(0, _POST, nms_step, 0)


def kernel(feats, ancs, ancs_valid, W1, b1, Wc, bc, Wr, br):
    del ancs_valid
    x = feats.reshape(_HW, 1024)

    wh = jnp.zeros((512, 128), jnp.float32)
    bh = jnp.zeros((128,), jnp.float32)
    wh = wh.at[:, 0:_NUM_ANCS].set(Wc)
    bh = bh.at[0:_NUM_ANCS].set(bc)
    for c in range(4):
        wh = wh.at[:, 16 * (c + 1):16 * (c + 1) + _NUM_ANCS].set(Wr[:, c::4])
        bh = bh.at[16 * (c + 1):16 * (c + 1) + _NUM_ANCS].set(br[c::4])

    y = pl.pallas_call(
        _mm_body,
        grid=(_HW // _TM,),
        in_specs=[
            pl.BlockSpec((_TM, 1024), lambda i: (i, 0)),
            pl.BlockSpec((1024, 512), lambda i: (0, 0)),
            pl.BlockSpec((1, 512), lambda i: (0, 0)),
            pl.BlockSpec((512, 128), lambda i: (0, 0)),
            pl.BlockSpec((1, 128), lambda i: (0, 0)),
        ],
        out_specs=pl.BlockSpec((_TM, 128), lambda i: (i, 0)),
        out_shape=jax.ShapeDtypeStruct((_HW, 128), jnp.float32),
    )(x, W1, b1.reshape(1, 512), wh, bh.reshape(1, 128))

    cls_pred = y[:, 0:_NUM_ANCS]
    reg_planes = [y[:, 16 * (c + 1):16 * (c + 1) + _NUM_ANCS] for c in range(4)]

    scores = cls_pred.reshape(_ROWS, 128)
    ancs_flat = ancs.reshape(_N, 4)
    anc_planes = [ancs_flat[:, c].reshape(_ROWS, 128) for c in range(4)]
    reg2 = [p.reshape(_ROWS, 128) for p in reg_planes]

    plane = jax.ShapeDtypeStruct((_ROWS, 128), jnp.float32)
    x1p, y1p, x2p, y2p, dst = pl.pallas_call(
        _prep_body,
        in_specs=[pl.BlockSpec(memory_space=pltpu.VMEM)] * 9,
        out_specs=[pl.BlockSpec(memory_space=pltpu.VMEM)] * 5,
        out_shape=[plane] * 4 + [jax.ShapeDtypeStruct((_ROWS, 128), jnp.int32)],
    )(scores, *anc_planes, *reg2)

    mesh = plsc.VectorSubcoreMesh(core_axis_name="c", subcore_axis_name="s")
    flat = jax.ShapeDtypeStruct((_CAP + _N,), jnp.float32)
    sc_compact = pl.kernel(
        _sc_compact_body, mesh=mesh,
        out_type=[flat] * 5,
        scratch_types=(
            [pltpu.VMEM((_CHUNK,), jnp.float32)] * 5
            + [pltpu.VMEM((_CHUNK,), jnp.int32),
               pltpu.SemaphoreType.DMA]
        ),
    )
    comp = sc_compact(scores.reshape(_N), x1p.reshape(_N), y1p.reshape(_N),
                      x2p.reshape(_N), y2p.reshape(_N), dst.reshape(_N))
    cplanes = [a[:_CAP].reshape(_CROWS, 128) for a in comp]

    outs = pl.pallas_call(
        _nms_body,
        in_specs=[pl.BlockSpec(memory_space=pltpu.VMEM)] * 5,
        out_specs=[pl.BlockSpec(memory_space=pltpu.SMEM)] * 4,
        out_shape=[jax.ShapeDtypeStruct((_POST,), jnp.float32)] * 4,
        scratch_shapes=[pltpu.VMEM((_CROWS, 128), jnp.float32)] * 2,
    )(*cplanes)

    proposals = jnp.stack(outs, axis=-1)
    cls_out = cls_pred.reshape(1, 64, 64, _NUM_ANCS)
    reg_out = jnp.stack(reg_planes, axis=-1).reshape(1, 64, 64, _NUM_ANCS, 4)
    return cls_out, reg_out, proposals


# speedup vs baseline: 2.7278x; 2.7093x over previous
import jax
import jax.numpy as jnp
from jax import lax
from jax.experimental import pallas as pl
from jax.experimental.pallas import tpu as pltpu

_NUM_ANCS = 9
_PRE = 6000
_POST = 300
_IOU_T = 0.7
_HW = 4096
_N = _HW * _NUM_ANCS
_ROWS = _N // 128
_TM = 512


def _mm_body(x_ref, w1_ref, b1_ref, wh_ref, bh_ref, y_ref):
    h = jnp.dot(x_ref[...], w1_ref[...], preferred_element_type=jnp.float32)
    h = jnp.maximum(h + b1_ref[...], 0.0)
    y = jnp.dot(h, wh_ref[...], preferred_element_type=jnp.float32) + bh_ref[...]
    lane = lax.broadcasted_iota(jnp.int32, (_TM, 128), 1)
    y_ref[...] = jnp.where(lane < 16, jax.nn.sigmoid(y), y)


def _nms_body(s_ref, a0_ref, a1_ref, a2_ref, a3_ref, r0_ref, r1_ref, r2_ref,
              r3_ref, ox1_ref, oy1_ref, ox2_ref, oy2_ref,
              ms_ref, x1_ref, y1_ref, x2_ref, y2_ref, ar_ref):
    a2 = a2_ref[...]
    a3 = a3_ref[...]
    cx = a0_ref[...] + r0_ref[...] * a2
    cy = a1_ref[...] + r1_ref[...] * a3
    w = a2 * jnp.exp(r2_ref[...])
    h = a3 * jnp.exp(r3_ref[...])
    x1 = jnp.clip(cx - w * 0.5, 0.0, 1.0)
    y1 = jnp.clip(cy - h * 0.5, 0.0, 1.0)
    x2 = jnp.clip(cx + w * 0.5, 0.0, 1.0)
    y2 = jnp.clip(cy + h * 0.5, 0.0, 1.0)
    x1_ref[...] = x1
    y1_ref[...] = y1
    x2_ref[...] = x2
    y2_ref[...] = y2
    ar_ref[...] = (x2 - x1) * (y2 - y1)

    s = s_ref[...]
    bits = lax.bitcast_convert_type(s, jnp.int32)
    idx = (lax.broadcasted_iota(jnp.int32, (_ROWS, 128), 0) * 128
           + lax.broadcasted_iota(jnp.int32, (_ROWS, 128), 1))
    tau = jnp.int32(0)
    for b in range(30, -1, -1):
        cand = tau | jnp.int32(1 << b)
        cnt = jnp.sum((bits >= cand).astype(jnp.int32))
        tau = jnp.where(cnt >= _PRE, cand, tau)
    cnt_gt = jnp.sum((bits > tau).astype(jnp.int32))
    tie = bits == tau
    need = _PRE - cnt_gt
    theta = jnp.int32(0)
    for b in range(15, -1, -1):
        cand = theta | jnp.int32(1 << b)
        cnt = jnp.sum((tie & (idx >= cand)).astype(jnp.int32))
        theta = jnp.where(cnt >= need, cand, theta)
    active = (bits > tau) | (tie & (idx >= theta))
    ms_ref[...] = jnp.where(active, s, -1.0)

    def zero_body(k, c):
        ox1_ref[k] = 0.0
        oy1_ref[k] = 0.0
        ox2_ref[k] = 0.0
        oy2_ref[k] = 0.0
        return c

    lax.fori_loop(0, _POST, zero_body, 0)

    ngroups = _ROWS // 8
    base_iota = (lax.broadcasted_iota(jnp.int32, (8, 128), 0) * 128
                 + lax.broadcasted_iota(jnp.int32, (8, 128), 1))
    lane_iota = lax.broadcasted_iota(jnp.int32, (1, 128), 1)

    def _argpair(vmax, vidx):
        m = jnp.max(vmax)
        bi = jnp.max(jnp.where(vmax == m, vidx, -1))
        return m, bi

    vmax0 = jnp.full((8, 128), -2.0, jnp.float32)
    vidx0 = jnp.full((8, 128), -1, jnp.int32)
    vmax, vidx = vmax0, vidx0
    for g in range(ngroups):
        v = ms_ref[pl.ds(g * 8, 8), :]
        ge = v >= vmax
        vmax = jnp.maximum(vmax, v)
        vidx = jnp.where(ge, base_iota + g * 1024, vidx)
    carry0 = _argpair(vmax, vidx)

    def nms_step(k, carry):
        m, bi = carry

        def selected(_):
            br_ = bi // 128
            bc_ = bi % 128
            lane_eq = lane_iota == bc_

            def _pick(ref):
                return jnp.sum(jnp.where(lane_eq, ref[pl.ds(br_, 1), :], 0.0))

            bx1 = _pick(x1_ref)
            by1 = _pick(y1_ref)
            bx2 = _pick(x2_ref)
            by2 = _pick(y2_ref)
            barea = (bx2 - bx1) * (by2 - by1)
            vmax, vidx = vmax0, vidx0
            for g in range(ngroups):
                sl = pl.ds(g * 8, 8)
                ix1 = jnp.maximum(x1_ref[sl, :], bx1)
                iy1 = jnp.maximum(y1_ref[sl, :], by1)
                ix2 = jnp.minimum(x2_ref[sl, :], bx2)
                iy2 = jnp.minimum(y2_ref[sl, :], by2)
                inter = (jnp.maximum(ix2 - ix1, 0.0)
                         * jnp.maximum(iy2 - iy1, 0.0))
                union = jnp.maximum(ar_ref[sl, :] - inter + barea, 1e-8)
                supp = inter > _IOU_T * union
                newv = jnp.where(supp, -1.0, ms_ref[sl, :])
                ms_ref[sl, :] = newv
                ge = newv >= vmax
                vmax = jnp.maximum(vmax, newv)
                vidx = jnp.where(ge, base_iota + g * 1024, vidx)
            ox1_ref[k] = bx1
            oy1_ref[k] = by1
            ox2_ref[k] = bx2
            oy2_ref[k] = by2
            return _argpair(vmax, vidx)

        return lax.cond(m >= 0.0, selected, lambda _: (m, bi), 0)

    lax.fori_loop(0, _POST, nms_step, carry0)


def kernel(feats, ancs, ancs_valid, W1, b1, Wc, bc, Wr, br):
    del ancs_valid
    x = feats.reshape(_HW, 1024)

    wh = jnp.zeros((512, 128), jnp.float32)
    bh = jnp.zeros((128,), jnp.float32)
    wh = wh.at[:, 0:_NUM_ANCS].set(Wc)
    bh = bh.at[0:_NUM_ANCS].set(bc)
    for c in range(4):
        wh = wh.at[:, 16 * (c + 1):16 * (c + 1) + _NUM_ANCS].set(Wr[:, c::4])
        bh = bh.at[16 * (c + 1):16 * (c + 1) + _NUM_ANCS].set(br[c::4])

    y = pl.pallas_call(
        _mm_body,
        grid=(_HW // _TM,),
        in_specs=[
            pl.BlockSpec((_TM, 1024), lambda i: (i, 0)),
            pl.BlockSpec((1024, 512), lambda i: (0, 0)),
            pl.BlockSpec((1, 512), lambda i: (0, 0)),
            pl.BlockSpec((512, 128), lambda i: (0, 0)),
            pl.BlockSpec((1, 128), lambda i: (0, 0)),
        ],
        out_specs=pl.BlockSpec((_TM, 128), lambda i: (i, 0)),
        out_shape=jax.ShapeDtypeStruct((_HW, 128), jnp.float32),
    )(x, W1, b1.reshape(1, 512), wh, bh.reshape(1, 128))

    cls_pred = y[:, 0:_NUM_ANCS]
    reg_planes = [y[:, 16 * (c + 1):16 * (c + 1) + _NUM_ANCS] for c in range(4)]

    scores = cls_pred.reshape(_ROWS, 128)
    ancs_flat = ancs.reshape(_N, 4)
    anc_planes = [ancs_flat[:, c].reshape(_ROWS, 128) for c in range(4)]
    reg2 = [p.reshape(_ROWS, 128) for p in reg_planes]

    outs = pl.pallas_call(
        _nms_body,
        in_specs=[pl.BlockSpec(memory_space=pltpu.VMEM)] * 9,
        out_specs=[pl.BlockSpec(memory_space=pltpu.SMEM)] * 4,
        out_shape=[jax.ShapeDtypeStruct((_POST,), jnp.float32)] * 4,
        scratch_shapes=[pltpu.VMEM((_ROWS, 128), jnp.float32)] * 6,
    )(scores, *anc_planes, *reg2)

    proposals = jnp.stack(outs, axis=-1)
    cls_out = cls_pred.reshape(1, 64, 64, _NUM_ANCS)
    reg_out = jnp.stack(reg_planes, axis=-1).reshape(1, 64, 64, _NUM_ANCS, 4)
    return cls_out, reg_out, proposals


# speedup vs baseline: 2.7336x; 1.0021x over previous
import jax
import jax.numpy as jnp
from jax import lax
from jax.experimental import pallas as pl
from jax.experimental.pallas import tpu as pltpu

_NUM_ANCS = 9
_PRE = 6000
_POST = 300
_IOU_T = 0.7
_HW = 4096
_N = _HW * _NUM_ANCS
_ROWS = _N // 128
_TM = 512


def _mm_body(x_ref, w1_ref, b1_ref, wh_ref, bh_ref, y_ref):
    h = jnp.dot(x_ref[...], w1_ref[...], preferred_element_type=jnp.float32)
    h = jnp.maximum(h + b1_ref[...], 0.0)
    y = jnp.dot(h, wh_ref[...], preferred_element_type=jnp.float32) + bh_ref[...]
    lane = lax.broadcasted_iota(jnp.int32, (_TM, 128), 1)
    y_ref[...] = jnp.where(lane < 16, jax.nn.sigmoid(y), y)


def _nms_body(s_ref, a0_ref, a1_ref, a2_ref, a3_ref, r0_ref, r1_ref, r2_ref,
              r3_ref, ox1_ref, oy1_ref, ox2_ref, oy2_ref,
              ms_ref, x1_ref, y1_ref, x2_ref, y2_ref, ar_ref):
    a2 = a2_ref[...]
    a3 = a3_ref[...]
    cx = a0_ref[...] + r0_ref[...] * a2
    cy = a1_ref[...] + r1_ref[...] * a3
    w = a2 * jnp.exp(r2_ref[...])
    h = a3 * jnp.exp(r3_ref[...])
    x1 = jnp.clip(cx - w * 0.5, 0.0, 1.0)
    y1 = jnp.clip(cy - h * 0.5, 0.0, 1.0)
    x2 = jnp.clip(cx + w * 0.5, 0.0, 1.0)
    y2 = jnp.clip(cy + h * 0.5, 0.0, 1.0)
    x1_ref[...] = x1
    y1_ref[...] = y1
    x2_ref[...] = x2
    y2_ref[...] = y2
    ar_ref[...] = (x2 - x1) * (y2 - y1)

    s = s_ref[...]
    bits = lax.bitcast_convert_type(s, jnp.int32)
    idx = (lax.broadcasted_iota(jnp.int32, (_ROWS, 128), 0) * 128
           + lax.broadcasted_iota(jnp.int32, (_ROWS, 128), 1))
    tau = jnp.int32(0)
    for b in range(30, -1, -1):
        cand = tau | jnp.int32(1 << b)
        cnt = jnp.sum((bits >= cand).astype(jnp.int32))
        tau = jnp.where(cnt >= _PRE, cand, tau)
    cnt_gt = jnp.sum((bits > tau).astype(jnp.int32))
    tie = bits == tau
    need = _PRE - cnt_gt
    theta = jnp.int32(0)
    for b in range(15, -1, -1):
        cand = theta | jnp.int32(1 << b)
        cnt = jnp.sum((tie & (idx >= cand)).astype(jnp.int32))
        theta = jnp.where(cnt >= need, cand, theta)
    active = (bits > tau) | (tie & (idx >= theta))
    ms_ref[...] = jnp.where(active, s, -1.0)

    def zero_body(k, c):
        ox1_ref[k] = 0.0
        oy1_ref[k] = 0.0
        ox2_ref[k] = 0.0
        oy2_ref[k] = 0.0
        return c

    lax.fori_loop(0, _POST, zero_body, 0)

    ngroups = _ROWS // 8
    base_iota = (lax.broadcasted_iota(jnp.int32, (8, 128), 0) * 128
                 + lax.broadcasted_iota(jnp.int32, (8, 128), 1))
    vmax0 = jnp.full((8, 128), -2.0, jnp.float32)
    vidx0 = jnp.full((8, 128), -1, jnp.int32)
    vc0 = jnp.zeros((8, 128), jnp.float32)

    def _traverse(suppress, bx1, by1, bx2, by2):
        barea = (bx2 - bx1) * (by2 - by1)
        vmax, vidx = vmax0, vidx0
        vx1, vy1, vx2, vy2 = vc0, vc0, vc0, vc0
        for g in range(ngroups):
            sl = pl.ds(g * 8, 8)
            x1g = x1_ref[sl, :]
            y1g = y1_ref[sl, :]
            x2g = x2_ref[sl, :]
            y2g = y2_ref[sl, :]
            if suppress:
                ix1 = jnp.maximum(x1g, bx1)
                iy1 = jnp.maximum(y1g, by1)
                ix2 = jnp.minimum(x2g, bx2)
                iy2 = jnp.minimum(y2g, by2)
                inter = (jnp.maximum(ix2 - ix1, 0.0)
                         * jnp.maximum(iy2 - iy1, 0.0))
                union = jnp.maximum(ar_ref[sl, :] - inter + barea, 1e-8)
                supp = inter > _IOU_T * union
                newv = jnp.where(supp, -1.0, ms_ref[sl, :])
                ms_ref[sl, :] = newv
            else:
                newv = ms_ref[sl, :]
            ge = newv >= vmax
            vmax = jnp.maximum(vmax, newv)
            vidx = jnp.where(ge, base_iota + g * 1024, vidx)
            vx1 = jnp.where(ge, x1g, vx1)
            vy1 = jnp.where(ge, y1g, vy1)
            vx2 = jnp.where(ge, x2g, vx2)
            vy2 = jnp.where(ge, y2g, vy2)
        m = jnp.max(vmax)
        eqm = vmax == m
        bi = jnp.max(jnp.where(eqm, vidx, -1))
        eqi = eqm & (vidx == bi)
        ex1 = jnp.max(jnp.where(eqi, vx1, -2.0))
        ey1 = jnp.max(jnp.where(eqi, vy1, -2.0))
        ex2 = jnp.max(jnp.where(eqi, vx2, -2.0))
        ey2 = jnp.max(jnp.where(eqi, vy2, -2.0))
        return m, bi, ex1, ey1, ex2, ey2

    zf = jnp.float32(0.0)
    carry0 = _traverse(False, zf, zf, zf, zf)

    def nms_step(k, carry):
        m, bi, bx1, by1, bx2, by2 = carry

        def selected(_):
            ox1_ref[k] = bx1
            oy1_ref[k] = by1
            ox2_ref[k] = bx2
            oy2_ref[k] = by2
            return _traverse(True, bx1, by1, bx2, by2)

        return lax.cond(m >= 0.0, selected, lambda _: carry, 0)

    lax.fori_loop(0, _POST, nms_step, carry0)


def kernel(feats, ancs, ancs_valid, W1, b1, Wc, bc, Wr, br):
    del ancs_valid
    x = feats.reshape(_HW, 1024)

    wh = jnp.zeros((512, 128), jnp.float32)
    bh = jnp.zeros((128,), jnp.float32)
    wh = wh.at[:, 0:_NUM_ANCS].set(Wc)
    bh = bh.at[0:_NUM_ANCS].set(bc)
    for c in range(4):
        wh = wh.at[:, 16 * (c + 1):16 * (c + 1) + _NUM_ANCS].set(Wr[:, c::4])
        bh = bh.at[16 * (c + 1):16 * (c + 1) + _NUM_ANCS].set(br[c::4])

    y = pl.pallas_call(
        _mm_body,
        grid=(_HW // _TM,),
        in_specs=[
            pl.BlockSpec((_TM, 1024), lambda i: (i, 0)),
            pl.BlockSpec((1024, 512), lambda i: (0, 0)),
            pl.BlockSpec((1, 512), lambda i: (0, 0)),
            pl.BlockSpec((512, 128), lambda i: (0, 0)),
            pl.BlockSpec((1, 128), lambda i: (0, 0)),
        ],
        out_specs=pl.BlockSpec((_TM, 128), lambda i: (i, 0)),
        out_shape=jax.ShapeDtypeStruct((_HW, 128), jnp.float32),
    )(x, W1, b1.reshape(1, 512), wh, bh.reshape(1, 128))

    cls_pred = y[:, 0:_NUM_ANCS]
    reg_planes = [y[:, 16 * (c + 1):16 * (c + 1) + _NUM_ANCS] for c in range(4)]

    scores = cls_pred.reshape(_ROWS, 128)
    ancs_flat = ancs.reshape(_N, 4)
    anc_planes = [ancs_flat[:, c].reshape(_ROWS, 128) for c in range(4)]
    reg2 = [p.reshape(_ROWS, 128) for p in reg_planes]

    outs = pl.pallas_call(
        _nms_body,
        in_specs=[pl.BlockSpec(memory_space=pltpu.VMEM)] * 9,
        out_specs=[pl.BlockSpec(memory_space=pltpu.SMEM)] * 4,
        out_shape=[jax.ShapeDtypeStruct((_POST,), jnp.float32)] * 4,
        scratch_shapes=[pltpu.VMEM((_ROWS, 128), jnp.float32)] * 6,
    )(scores, *anc_planes, *reg2)

    proposals = jnp.stack(outs, axis=-1)
    cls_out = cls_pred.reshape(1, 64, 64, _NUM_ANCS)
    reg_out = jnp.stack(reg_planes, axis=-1).reshape(1, 64, 64, _NUM_ANCS, 4)
    return cls_out, reg_out, proposals


# speedup vs baseline: 2.7881x; 1.0199x over previous
import jax
import jax.numpy as jnp
from jax import lax
from jax.experimental import pallas as pl
from jax.experimental.pallas import tpu as pltpu

_NUM_ANCS = 9
_PRE = 6000
_POST = 300
_IOU_T = 0.7
_HW = 4096
_N = _HW * _NUM_ANCS
_ROWS = _N // 128
_TM = 512


def _mm_body(x_ref, w1_ref, b1_ref, wh_ref, bh_ref, y_ref):
    h = jnp.dot(x_ref[...], w1_ref[...], preferred_element_type=jnp.float32)
    h = jnp.maximum(h + b1_ref[...], 0.0)
    y = jnp.dot(h, wh_ref[...], preferred_element_type=jnp.float32) + bh_ref[...]
    lane = lax.broadcasted_iota(jnp.int32, (_TM, 128), 1)
    y_ref[...] = jnp.where(lane < 16, jax.nn.sigmoid(y), y)


def _nms_body(s_ref, a0_ref, a1_ref, a2_ref, a3_ref, r0_ref, r1_ref, r2_ref,
              r3_ref, ox1_ref, oy1_ref, ox2_ref, oy2_ref,
              ms_ref, x1_ref, y1_ref, x2_ref, y2_ref, ar_ref):
    a2 = a2_ref[...]
    a3 = a3_ref[...]
    cx = a0_ref[...] + r0_ref[...] * a2
    cy = a1_ref[...] + r1_ref[...] * a3
    w = a2 * jnp.exp(r2_ref[...])
    h = a3 * jnp.exp(r3_ref[...])
    x1 = jnp.clip(cx - w * 0.5, 0.0, 1.0)
    y1 = jnp.clip(cy - h * 0.5, 0.0, 1.0)
    x2 = jnp.clip(cx + w * 0.5, 0.0, 1.0)
    y2 = jnp.clip(cy + h * 0.5, 0.0, 1.0)
    x1_ref[...] = x1
    y1_ref[...] = y1
    x2_ref[...] = x2
    y2_ref[...] = y2
    ar_ref[...] = (x2 - x1) * (y2 - y1)

    s = s_ref[...]
    bits = lax.bitcast_convert_type(s, jnp.int32)
    idx = (lax.broadcasted_iota(jnp.int32, (_ROWS, 128), 0) * 128
           + lax.broadcasted_iota(jnp.int32, (_ROWS, 128), 1))
    tau = jnp.int32(0)
    for b in range(30, -1, -1):
        cand = tau | jnp.int32(1 << b)
        cnt = jnp.sum((bits >= cand).astype(jnp.int32))
        tau = jnp.where(cnt >= _PRE, cand, tau)
    cnt_gt = jnp.sum((bits > tau).astype(jnp.int32))
    tie = bits == tau
    need = _PRE - cnt_gt
    theta = jnp.int32(0)
    for b in range(15, -1, -1):
        cand = theta | jnp.int32(1 << b)
        cnt = jnp.sum((tie & (idx >= cand)).astype(jnp.int32))
        theta = jnp.where(cnt >= need, cand, theta)
    active = (bits > tau) | (tie & (idx >= theta))
    ms_ref[...] = jnp.where(active, s, -1.0)

    ngroups = _ROWS // 8
    base_iota = (lax.broadcasted_iota(jnp.int32, (8, 128), 0) * 128
                 + lax.broadcasted_iota(jnp.int32, (8, 128), 1))
    vmax0 = jnp.full((8, 128), -2.0, jnp.float32)
    vidx0 = jnp.full((8, 128), -1, jnp.int32)
    vc0 = jnp.zeros((8, 128), jnp.float32)

    def _traverse(suppress, bx1, by1, bx2, by2):
        barea = (bx2 - bx1) * (by2 - by1)
        vmax, vidx = vmax0, vidx0
        vx1, vy1, vx2, vy2 = vc0, vc0, vc0, vc0
        for g in range(ngroups):
            sl = pl.ds(g * 8, 8)
            x1g = x1_ref[sl, :]
            y1g = y1_ref[sl, :]
            x2g = x2_ref[sl, :]
            y2g = y2_ref[sl, :]
            if suppress:
                ix1 = jnp.maximum(x1g, bx1)
                iy1 = jnp.maximum(y1g, by1)
                ix2 = jnp.minimum(x2g, bx2)
                iy2 = jnp.minimum(y2g, by2)
                inter = (jnp.maximum(ix2 - ix1, 0.0)
                         * jnp.maximum(iy2 - iy1, 0.0))
                union = jnp.maximum(ar_ref[sl, :] - inter + barea, 1e-8)
                supp = inter > _IOU_T * union
                newv = jnp.where(supp, -1.0, ms_ref[sl, :])
                ms_ref[sl, :] = newv
            else:
                newv = ms_ref[sl, :]
            ge = newv >= vmax
            vmax = jnp.maximum(vmax, newv)
            vidx = jnp.where(ge, base_iota + g * 1024, vidx)
            vx1 = jnp.where(ge, x1g, vx1)
            vy1 = jnp.where(ge, y1g, vy1)
            vx2 = jnp.where(ge, x2g, vx2)
            vy2 = jnp.where(ge, y2g, vy2)
        m = jnp.max(vmax)
        eqm = vmax == m
        bi = jnp.max(jnp.where(eqm, vidx, -1))
        eqi = eqm & (vidx == bi)
        ex1 = jnp.max(jnp.where(eqi, vx1, -2.0))
        ey1 = jnp.max(jnp.where(eqi, vy1, -2.0))
        ex2 = jnp.max(jnp.where(eqi, vx2, -2.0))
        ey2 = jnp.max(jnp.where(eqi, vy2, -2.0))
        return m, bi, ex1, ey1, ex2, ey2

    zf = jnp.float32(0.0)
    carry0 = _traverse(False, zf, zf, zf, zf)

    def nms_step(k, carry):
        m, bi, bx1, by1, bx2, by2 = carry
        valid = m >= 0.0
        ox1_ref[k] = jnp.where(valid, bx1, 0.0)
        oy1_ref[k] = jnp.where(valid, by1, 0.0)
        ox2_ref[k] = jnp.where(valid, bx2, 0.0)
        oy2_ref[k] = jnp.where(valid, by2, 0.0)
        return _traverse(True, bx1, by1, bx2, by2)

    lax.fori_loop(0, _POST, nms_step, carry0)


def kernel(feats, ancs, ancs_valid, W1, b1, Wc, bc, Wr, br):
    del ancs_valid
    x = feats.reshape(_HW, 1024)

    wh = jnp.zeros((512, 128), jnp.float32)
    bh = jnp.zeros((128,), jnp.float32)
    wh = wh.at[:, 0:_NUM_ANCS].set(Wc)
    bh = bh.at[0:_NUM_ANCS].set(bc)
    for c in range(4):
        wh = wh.at[:, 16 * (c + 1):16 * (c + 1) + _NUM_ANCS].set(Wr[:, c::4])
        bh = bh.at[16 * (c + 1):16 * (c + 1) + _NUM_ANCS].set(br[c::4])

    y = pl.pallas_call(
        _mm_body,
        grid=(_HW // _TM,),
        in_specs=[
            pl.BlockSpec((_TM, 1024), lambda i: (i, 0)),
            pl.BlockSpec((1024, 512), lambda i: (0, 0)),
            pl.BlockSpec((1, 512), lambda i: (0, 0)),
            pl.BlockSpec((512, 128), lambda i: (0, 0)),
            pl.BlockSpec((1, 128), lambda i: (0, 0)),
        ],
        out_specs=pl.BlockSpec((_TM, 128), lambda i: (i, 0)),
        out_shape=jax.ShapeDtypeStruct((_HW, 128), jnp.float32),
    )(x, W1, b1.reshape(1, 512), wh, bh.reshape(1, 128))

    cls_pred = y[:, 0:_NUM_ANCS]
    reg_planes = [y[:, 16 * (c + 1):16 * (c + 1) + _NUM_ANCS] for c in range(4)]

    scores = cls_pred.reshape(_ROWS, 128)
    ancs_flat = ancs.reshape(_N, 4)
    anc_planes = [ancs_flat[:, c].reshape(_ROWS, 128) for c in range(4)]
    reg2 = [p.reshape(_ROWS, 128) for p in reg_planes]

    outs = pl.pallas_call(
        _nms_body,
        in_specs=[pl.BlockSpec(memory_space=pltpu.VMEM)] * 9,
        out_specs=[pl.BlockSpec(memory_space=pltpu.SMEM)] * 4,
        out_shape=[jax.ShapeDtypeStruct((_POST,), jnp.float32)] * 4,
        scratch_shapes=[pltpu.VMEM((_ROWS, 128), jnp.float32)] * 6,
    )(scores, *anc_planes, *reg2)

    proposals = jnp.stack(outs, axis=-1)
    cls_out = cls_pred.reshape(1, 64, 64, _NUM_ANCS)
    reg_out = jnp.stack(reg_planes, axis=-1).reshape(1, 64, 64, _NUM_ANCS, 4)
    return cls_out, reg_out, proposals
